# baseline (speedup 1.0000x reference)
import functools

import jax
import jax.numpy as jnp
from jax import lax
from jax.experimental import pallas as pl
from jax.experimental.pallas import tpu as pltpu
from jax.experimental.pallas import tpu_sc as plsc

N = 100000
E = 1600000
DIN = 21
DH = 16
DOUT = 6
DHALF = DH // 2

NC = 2
NS = 16
NW = NC * NS
CHUNK = 128
NCHT = 12512
EPAD = NCHT * CHUNK
NPAD = EPAD - E
CPW32 = NCHT // NW
CPW16 = NCHT // NS
G = 23
NSC32 = CPW32 // G
NSC16 = CPW16 // G
NTRASH = 16
NROWS = 100096
RPT = NROWS // NS

ROWB = 4000
GRID = N // ROWB


def _sc_mesh():
    return plsc.VectorSubcoreMesh(core_axis_name="c", subcore_axis_name="s")


_SC_PARAMS = pltpu.CompilerParams(use_tc_tiling_on_sc=False,
                                  needs_layout_passes=False)


@functools.partial(
    pl.kernel,
    out_type=jax.ShapeDtypeStruct((GRID, NW, ROWB), jnp.float32),
    mesh=_sc_mesh(),
    scratch_types=[
        pltpu.VMEM((N + NTRASH,), jnp.float32),
        pltpu.VMEM((G, CHUNK), jnp.int32),
    ],
    compiler_params=_SC_PARAMS,
)
def _deg_kernel(dst_hbm, out, hist, dbuf):
    c = lax.axis_index("c")
    s = lax.axis_index("s")
    wid = c * NS + s

    def zero_body(i, carry):
        hist[pl.ds(i * 16, 16)] = jnp.zeros((16,), jnp.float32)
        return carry

    lax.fori_loop(0, (N + NTRASH) // 16, zero_body, 0)

    ones = jnp.ones((16,), jnp.float32)

    def body(sc, carry):
        pltpu.sync_copy(dst_hbm.at[pl.ds(wid * CPW32 + sc * G, G)], dbuf)
        for k in range(G):
            for g in range(CHUNK // 16):
                idxv = dbuf[k, pl.ds(g * 16, 16)]
                plsc.addupdate_scatter(hist, [idxv], ones)
        return carry

    lax.fori_loop(0, NSC32, body, 0)
    for gi in range(GRID):
        pltpu.sync_copy(hist.at[pl.ds(gi * ROWB, ROWB)], out.at[gi, wid])


@functools.partial(
    pl.kernel,
    out_type=jax.ShapeDtypeStruct((NC, NROWS, DHALF), jnp.float32),
    mesh=_sc_mesh(),
    scratch_types=[
        pltpu.VMEM((G, CHUNK), jnp.int32),
        pltpu.VMEM((G, CHUNK), jnp.int32),
        pltpu.VMEM((G, CHUNK, DHALF), jnp.float32),
        pltpu.VMEM_SHARED((NROWS, DHALF), jnp.float32),
        pltpu.SemaphoreType.DMA,
    ],
    compiler_params=_SC_PARAMS,
)
def _msg_kernel(table_hbm, src_hbm, dst_hbm, zeros_hbm, out, sbuf, dbuf,
                rows_v, acc, gsem):
    c = lax.axis_index("c")
    s = lax.axis_index("s")
    pltpu.sync_copy(zeros_hbm, acc.at[pl.ds(s * RPT, RPT)])
    plsc.subcore_barrier()
    table = table_hbm.at[c]

    def body(sc, carry):
        base = s * CPW16 + sc * G
        pltpu.sync_copy(src_hbm.at[pl.ds(base, G)], sbuf)
        pltpu.sync_copy(dst_hbm.at[pl.ds(base, G)], dbuf)
        gd = [pltpu.async_copy(table.at[sbuf.at[k]], rows_v.at[k], gsem)
              for k in range(G)]
        for d in gd:
            d.wait()
        for k in range(G):
            pltpu.sync_copy(rows_v.at[k], acc.at[dbuf.at[k]], add=True)
        return carry

    lax.fori_loop(0, NSC16, body, 0)
    plsc.subcore_barrier()
    pltpu.sync_copy(acc.at[pl.ds(s * RPT, RPT)], out.at[c, pl.ds(s * RPT, RPT)])


_SUM_DN = (((0,), (0,)), ((), ()))


def _tc1_body(x_ref, w1_ref, hp_ref, onesw_ref, y1_ref, dis_ref):
    hp = jnp.reshape(hp_ref[...], (NW, ROWB))
    deg = lax.dot_general(hp, onesw_ref[...], _SUM_DN,
                          preferred_element_type=jnp.float32) + 1.0
    dis = lax.rsqrt(deg)
    xw = jnp.dot(x_ref[...], w1_ref[...], preferred_element_type=jnp.float32)
    y1_ref[...] = xw * dis
    dis_ref[...] = dis


def _tc2_body(a0_ref, a1_ref, y1_ref, dis_ref, b1_ref, w2_ref, y2_ref):
    dis = dis_ref[...]
    agg = jnp.concatenate([a0_ref[...], a1_ref[...]], axis=1)
    out1 = (agg + y1_ref[...]) * dis + b1_ref[...]
    h = jnp.maximum(out1, 0.0)
    y2_ref[...] = jnp.dot(h, w2_ref[...], preferred_element_type=jnp.float32) * dis


def _tc3_body(a0_ref, a1_ref, y2_ref, dis_ref, b2_ref, out_ref):
    agg = jnp.concatenate([a0_ref[...], a1_ref[...]], axis=1)
    z = (agg + y2_ref[...]) * dis_ref[...] + b2_ref[...]
    col = lax.broadcasted_iota(jnp.int32, z.shape, 1)
    valid = col < DOUT
    m = jnp.max(jnp.where(valid, z, -1e30), axis=1, keepdims=True)
    ex = jnp.where(valid, jnp.exp(z - m), 0.0)
    lse = jnp.log(jnp.sum(ex, axis=1, keepdims=True)) + m
    out_ref[...] = (z - lse)[:, :DOUT]


def _row_spec(width):
    return pl.BlockSpec((ROWB, width), lambda i: (i, 0))


def _full_spec(shape):
    return pl.BlockSpec(shape, lambda i: tuple(0 for _ in shape))


_tc1 = pl.pallas_call(
    _tc1_body,
    grid=(GRID,),
    in_specs=[_row_spec(DIN), _full_spec((DIN, DH)),
              pl.BlockSpec((1, NW, ROWB), lambda i: (i, 0, 0)),
              _full_spec((NW, 1))],
    out_specs=[_row_spec(DH), _row_spec(1)],
    out_shape=[jax.ShapeDtypeStruct((N, DH), jnp.float32),
               jax.ShapeDtypeStruct((N, 1), jnp.float32)],
)

_tc2 = pl.pallas_call(
    _tc2_body,
    grid=(GRID,),
    in_specs=[_row_spec(DHALF), _row_spec(DHALF), _row_spec(DH), _row_spec(1),
              _full_spec((1, DH)), _full_spec((DH, DH))],
    out_specs=[_row_spec(DH)],
    out_shape=[jax.ShapeDtypeStruct((N, DH), jnp.float32)],
)

_tc3 = pl.pallas_call(
    _tc3_body,
    grid=(GRID,),
    in_specs=[_row_spec(DHALF), _row_spec(DHALF), _row_spec(DH), _row_spec(1),
              _full_spec((1, DH))],
    out_specs=[_row_spec(DOUT)],
    out_shape=[jax.ShapeDtypeStruct((N, DOUT), jnp.float32)],
)


def _split_table(y):
    yp = jnp.concatenate([y, jnp.zeros((1, DH), jnp.float32)], axis=0)
    return jnp.stack([yp[:, :DHALF], yp[:, DHALF:]], axis=0)


def kernel(x, edge_index, W1, b1, W2, b2):
    ei = edge_index.astype(jnp.int32)
    src = jnp.concatenate([ei[0], jnp.full((NPAD,), N, jnp.int32)])
    dst = jnp.concatenate([ei[1], jnp.full((NPAD,), N, jnp.int32)])
    srcr = src.reshape(NCHT, CHUNK)
    dstr = dst.reshape(NCHT, CHUNK)

    zeros8 = jnp.zeros((RPT, DHALF), jnp.float32)
    onesw = jnp.ones((NW, 1), jnp.float32)

    b1r = b1.reshape(1, DH)
    w2p = jnp.zeros((DH, DH), jnp.float32).at[:, :DOUT].set(W2)
    b2p = jnp.zeros((1, DH), jnp.float32).at[0, :DOUT].set(b2)

    histp = _deg_kernel(dstr)
    y1, dis = _tc1(x, W1, histp, onesw)

    acc1 = _msg_kernel(_split_table(y1), srcr, dstr, zeros8)
    (y2,) = _tc2(acc1[0], acc1[1], y1, dis, b1r, w2p)

    acc2 = _msg_kernel(_split_table(y2), srcr, dstr, zeros8)
    (out,) = _tc3(acc2[0], acc2[1], y2, dis, b2p)
    return out


# speedup vs baseline: 1.1865x; 1.1865x over previous
import functools

import jax
import jax.numpy as jnp
from jax import lax
from jax.experimental import pallas as pl
from jax.experimental.pallas import tpu as pltpu
from jax.experimental.pallas import tpu_sc as plsc

N = 100000
E = 1600000
DIN = 21
DH = 16
DOUT = 6
DHALF = DH // 2

NC = 2
NS = 16
NW = NC * NS
CHUNK = 128
NCHT = 12512
EPAD = NCHT * CHUNK
NPAD = EPAD - E
CPW32 = NCHT // NW
CPW16 = NCHT // NS
G = 23
NSC32 = CPW32 // G
NSC16 = CPW16 // G
NTRASH = 16
NROWS = 100096
RPT = NROWS // NS
N1 = NROWS + 8

ROWB = 4000
GRID = N // ROWB


def _sc_mesh():
    return plsc.VectorSubcoreMesh(core_axis_name="c", subcore_axis_name="s")


_SC_PARAMS = pltpu.CompilerParams(use_tc_tiling_on_sc=False,
                                  needs_layout_passes=False)


@functools.partial(
    pl.kernel,
    out_type=jax.ShapeDtypeStruct((GRID, NW, ROWB), jnp.float32),
    mesh=_sc_mesh(),
    scratch_types=[
        pltpu.VMEM((N + NTRASH,), jnp.float32),
        pltpu.VMEM((G, CHUNK), jnp.int32),
    ],
    compiler_params=_SC_PARAMS,
)
def _deg_kernel(dst_hbm, out, hist, dbuf):
    c = lax.axis_index("c")
    s = lax.axis_index("s")
    wid = c * NS + s

    def zero_body(i, carry):
        hist[pl.ds(i * 16, 16)] = jnp.zeros((16,), jnp.float32)
        return carry

    lax.fori_loop(0, (N + NTRASH) // 16, zero_body, 0)

    ones = jnp.ones((16,), jnp.float32)

    def body(sc, carry):
        pltpu.sync_copy(dst_hbm.at[pl.ds(wid * CPW32 + sc * G, G)], dbuf)
        for k in range(G):
            for g in range(CHUNK // 16):
                idxv = dbuf[k, pl.ds(g * 16, 16)]
                plsc.addupdate_scatter(hist, [idxv], ones)
        return carry

    lax.fori_loop(0, NSC32, body, 0)
    for gi in range(GRID):
        pltpu.sync_copy(hist.at[pl.ds(gi * ROWB, ROWB)], out.at[gi, wid])


@functools.partial(
    pl.kernel,
    out_type=jax.ShapeDtypeStruct((NC, NROWS, DHALF), jnp.float32),
    mesh=_sc_mesh(),
    scratch_types=[
        pltpu.VMEM((G, CHUNK), jnp.int32),
        pltpu.VMEM((G, CHUNK), jnp.int32),
        pltpu.VMEM((G, CHUNK, DHALF), jnp.float32),
        pltpu.VMEM_SHARED((NROWS, DHALF), jnp.float32),
        pltpu.SemaphoreType.DMA,
    ],
    compiler_params=_SC_PARAMS,
)
def _msg_kernel(table_hbm, src_hbm, dst_hbm, out, sbuf, dbuf,
                rows_v, acc, gsem):
    c = lax.axis_index("c")
    s = lax.axis_index("s")
    table = table_hbm.at[c]
    pltpu.sync_copy(table.at[pl.ds(s * RPT, RPT)], acc.at[pl.ds(s * RPT, RPT)])
    plsc.subcore_barrier()

    def body(sc, carry):
        base = s * CPW16 + sc * G
        pltpu.sync_copy(src_hbm.at[pl.ds(base, G)], sbuf)
        pltpu.sync_copy(dst_hbm.at[pl.ds(base, G)], dbuf)
        gd = [pltpu.async_copy(table.at[sbuf.at[k]], rows_v.at[k], gsem)
              for k in range(G)]
        for d in gd:
            d.wait()
        for k in range(G):
            pltpu.sync_copy(rows_v.at[k], acc.at[dbuf.at[k]], add=True)
        return carry

    lax.fori_loop(0, NSC16, body, 0)
    plsc.subcore_barrier()
    pltpu.sync_copy(acc.at[pl.ds(s * RPT, RPT)], out.at[c, pl.ds(s * RPT, RPT)])


_SUM_DN = (((0,), (0,)), ((), ()))


def _tc1_body(x_ref, w1_ref, hp_ref, onesw_ref, y1_ref, dis_ref):
    hp = jnp.reshape(hp_ref[...], (NW, ROWB))
    deg = lax.dot_general(hp, onesw_ref[...], _SUM_DN,
                          preferred_element_type=jnp.float32) + 1.0
    dis = lax.rsqrt(deg)
    xw = jnp.dot(x_ref[...], w1_ref[...], preferred_element_type=jnp.float32)
    y1_ref[...] = xw * dis
    dis_ref[...] = dis


def _tc2_body(agg_ref, dis_ref, b1_ref, w2_ref, y2_ref):
    dis = dis_ref[...]
    out1 = agg_ref[...] * dis + b1_ref[...]
    h = jnp.maximum(out1, 0.0)
    y2_ref[...] = jnp.dot(h, w2_ref[...], preferred_element_type=jnp.float32) * dis


def _tc3_body(agg_ref, dis_ref, b2_ref, out_ref):
    z = agg_ref[...] * dis_ref[...] + b2_ref[...]
    col = lax.broadcasted_iota(jnp.int32, z.shape, 1)
    valid = col < DOUT
    m = jnp.max(jnp.where(valid, z, -1e30), axis=1, keepdims=True)
    ex = jnp.where(valid, jnp.exp(z - m), 0.0)
    lse = jnp.log(jnp.sum(ex, axis=1, keepdims=True)) + m
    out_ref[...] = (z - lse)[:, :DOUT]


def _row_spec(width):
    return pl.BlockSpec((ROWB, width), lambda i: (i, 0))


def _full_spec(shape):
    return pl.BlockSpec(shape, lambda i: tuple(0 for _ in shape))


_tc1 = pl.pallas_call(
    _tc1_body,
    grid=(GRID,),
    in_specs=[_row_spec(DIN), _full_spec((DIN, DH)),
              pl.BlockSpec((1, NW, ROWB), lambda i: (i, 0, 0)),
              _full_spec((NW, 1))],
    out_specs=[_row_spec(DH), _row_spec(1)],
    out_shape=[jax.ShapeDtypeStruct((N, DH), jnp.float32),
               jax.ShapeDtypeStruct((N, 1), jnp.float32)],
)

_tc2 = pl.pallas_call(
    _tc2_body,
    grid=(GRID,),
    in_specs=[_row_spec(DH), _row_spec(1),
              _full_spec((1, DH)), _full_spec((DH, DH))],
    out_specs=[_row_spec(DH)],
    out_shape=[jax.ShapeDtypeStruct((N, DH), jnp.float32)],
)

_tc3 = pl.pallas_call(
    _tc3_body,
    grid=(GRID,),
    in_specs=[_row_spec(DH), _row_spec(1), _full_spec((1, DH))],
    out_specs=[_row_spec(DOUT)],
    out_shape=[jax.ShapeDtypeStruct((N, DOUT), jnp.float32)],
)


def _split_table(y):
    yp = jnp.concatenate([y, jnp.zeros((N1 - N, DH), jnp.float32)], axis=0)
    return jnp.swapaxes(yp.reshape(N1, NC, DHALF), 0, 1)


def _merge_acc(accp):
    return jnp.swapaxes(accp, 0, 1).reshape(NROWS, DH)


def kernel(x, edge_index, W1, b1, W2, b2):
    ei = edge_index.astype(jnp.int32)
    src = jnp.concatenate([ei[0], jnp.full((NPAD,), N, jnp.int32)])
    dst = jnp.concatenate([ei[1], jnp.full((NPAD,), N, jnp.int32)])
    srcr = src.reshape(NCHT, CHUNK)
    dstr = dst.reshape(NCHT, CHUNK)

    onesw = jnp.ones((NW, 1), jnp.float32)

    b1r = b1.reshape(1, DH)
    w2p = jnp.zeros((DH, DH), jnp.float32).at[:, :DOUT].set(W2)
    b2p = jnp.zeros((1, DH), jnp.float32).at[0, :DOUT].set(b2)

    histp = _deg_kernel(dstr)
    y1, dis = _tc1(x, W1, histp, onesw)

    acc1 = _msg_kernel(_split_table(y1), srcr, dstr)
    (y2,) = _tc2(_merge_acc(acc1), dis, b1r, w2p)

    acc2 = _msg_kernel(_split_table(y2), srcr, dstr)
    (out,) = _tc3(_merge_acc(acc2), dis, b2p)
    return out
